# baseline (speedup 1.0000x reference)
import functools

import jax
import jax.numpy as jnp
from jax import lax
from jax.experimental import pallas as pl
from jax.experimental.pallas import tpu as pltpu
from jax.experimental.pallas import tpu_sc as plsc

N = 10000
E = 320000
D_IN = 128
H = 32

NP = 10240
DUMMY_DST = N
DUMMY_SRC = N + 1

NW = 32
CH = 128
NCH = 84
NBUF = 4
EP = NW * NCH * CH
ROWS_PER_TILE = NP // 16


@functools.cache
def _sc_kernels():
    mesh = plsc.VectorSubcoreMesh(core_axis_name="c", subcore_axis_name="s")
    params = pltpu.CompilerParams(use_tc_tiling_on_sc=False)

    @functools.partial(
        pl.kernel,
        out_type=jax.ShapeDtypeStruct((2, NP), jnp.float32),
        mesh=mesh,
        compiler_params=params,
        scratch_types=[
            pltpu.VMEM((NCH, CH), jnp.int32),
            pltpu.VMEM((CH,), jnp.float32),
            pltpu.VMEM_SHARED((NP,), jnp.float32),
            [pltpu.SemaphoreType.DMA] * NBUF,
        ],
    )
    def deg_kernel(didx_hbm, ones_hbm, zeros1_hbm, deg_out, didx_v, ones_v,
                   dacc, ssem):
        cid = lax.axis_index("c")
        sid = lax.axis_index("s")
        wid = sid * 2 + cid
        lo = sid * ROWS_PER_TILE
        pltpu.sync_copy(didx_hbm.at[wid], didx_v)
        pltpu.sync_copy(ones_hbm, ones_v)
        pltpu.sync_copy(zeros1_hbm.at[pl.ds(lo, ROWS_PER_TILE)],
                        dacc.at[pl.ds(lo, ROWS_PER_TILE)])
        plsc.subcore_barrier()

        for b in range(NBUF):
            pltpu.async_copy(ones_v, dacc.at[didx_v.at[b]], ssem[b], add=True)

        def body(it, carry):
            j = it * NBUF
            for b in range(NBUF):
                pltpu.make_async_copy(ones_v, dacc.at[didx_v.at[j + b]],
                                      ssem[b]).wait()
                pltpu.async_copy(ones_v, dacc.at[didx_v.at[j + NBUF + b]],
                                 ssem[b], add=True)
            return carry

        lax.fori_loop(0, NCH // NBUF - 1, body, 0)
        for b in range(NBUF):
            pltpu.make_async_copy(ones_v, dacc.at[didx_v.at[NCH - NBUF + b]],
                                  ssem[b]).wait()
        plsc.subcore_barrier()
        pltpu.sync_copy(dacc.at[pl.ds(lo, ROWS_PER_TILE)],
                        deg_out.at[cid, pl.ds(lo, ROWS_PER_TILE)])

    @functools.partial(
        pl.kernel,
        out_type=jax.ShapeDtypeStruct((2, NP, H), jnp.float32),
        mesh=mesh,
        compiler_params=params,
        scratch_types=[
            pltpu.VMEM((NCH, CH), jnp.int32),
            pltpu.VMEM((NCH, CH), jnp.int32),
            [pltpu.VMEM((CH, H), jnp.float32)] * NBUF,
            pltpu.VMEM_SHARED((NP, H), jnp.float32),
            [pltpu.SemaphoreType.DMA] * NBUF,
            [pltpu.SemaphoreType.DMA] * NBUF,
        ],
    )
    def round_kernel(tab_hbm, sidx_hbm, didx_hbm, zeros2_hbm, out_hbm,
                     sidx_v, didx_v, bufs, acc, gsem, ssem):
        cid = lax.axis_index("c")
        sid = lax.axis_index("s")
        wid = sid * 2 + cid
        lo = sid * ROWS_PER_TILE
        pltpu.sync_copy(sidx_hbm.at[wid], sidx_v)
        pltpu.sync_copy(didx_hbm.at[wid], didx_v)
        pltpu.sync_copy(zeros2_hbm.at[pl.ds(lo, ROWS_PER_TILE)],
                        acc.at[pl.ds(lo, ROWS_PER_TILE)])
        plsc.subcore_barrier()

        for b in range(NBUF):
            pltpu.async_copy(tab_hbm.at[sidx_v.at[b]], bufs[b], gsem[b])

        def body(it, carry):
            j = it * NBUF
            for b in range(NBUF):
                pltpu.make_async_copy(tab_hbm.at[sidx_v.at[j + b]],
                                      bufs[b], gsem[b]).wait()
                pltpu.async_copy(bufs[b], acc.at[didx_v.at[j + b]],
                                 ssem[b], add=True)
            for b in range(NBUF):
                pltpu.make_async_copy(bufs[b], acc.at[didx_v.at[j + b]],
                                      ssem[b]).wait()
                pltpu.async_copy(tab_hbm.at[sidx_v.at[j + NBUF + b]],
                                 bufs[b], gsem[b])
            return carry

        lax.fori_loop(0, NCH // NBUF - 1, body, 0)
        j = NCH - NBUF
        for b in range(NBUF):
            pltpu.make_async_copy(tab_hbm.at[sidx_v.at[j + b]],
                                  bufs[b], gsem[b]).wait()
            pltpu.async_copy(bufs[b], acc.at[didx_v.at[j + b]],
                             ssem[b], add=True)
        for b in range(NBUF):
            pltpu.make_async_copy(bufs[b], acc.at[didx_v.at[j + b]],
                                  ssem[b]).wait()
        plsc.subcore_barrier()
        pltpu.sync_copy(acc.at[pl.ds(lo, ROWS_PER_TILE)],
                        out_hbm.at[cid, pl.ds(lo, ROWS_PER_TILE)])

    return deg_kernel, round_kernel


def _pre_body(x_ref, wi_ref, d0_ref, d1_ref, t_ref, dinv_ref):
    deg = d0_ref[...] + d1_ref[...]
    dinv = jnp.where(deg > 0, lax.rsqrt(jnp.maximum(deg, 1e-12)), 0.0)
    m0 = jnp.dot(x_ref[...], wi_ref[...], preferred_element_type=jnp.float32)
    t_ref[...] = m0 * dinv
    dinv_ref[...] = dinv


def _node_body(a0_ref, a1_ref, dinv_ref, b_ref, w_ref, t_ref):
    dinv = dinv_ref[...]
    h = jnp.maximum((a0_ref[...] + a1_ref[...]) * dinv + b_ref[...], 0.0)
    t_ref[...] = jnp.dot(h, w_ref[...], preferred_element_type=jnp.float32) * dinv


def _final_body(a0_ref, a1_ref, dinv_ref, b_ref, wd1_ref, bd1_ref,
                wd2_ref, bd2_ref, out_ref):
    h = jnp.maximum((a0_ref[...] + a1_ref[...]) * dinv_ref[...] + b_ref[...], 0.0)
    rows = lax.broadcasted_iota(jnp.int32, (NP, H), 0)
    hm = jnp.where(rows < N, h, -jnp.inf)
    z = jnp.max(hm, axis=0, keepdims=True)
    z2 = jnp.maximum(
        jnp.dot(z, wd1_ref[...], preferred_element_type=jnp.float32) + bd1_ref[...],
        0.0)
    out_ref[...] = (jnp.dot(z2, wd2_ref[...], preferred_element_type=jnp.float32)
                    + bd2_ref[...])


_pre_call = pl.pallas_call(
    _pre_body,
    out_shape=(jax.ShapeDtypeStruct((NP, H), jnp.float32),
               jax.ShapeDtypeStruct((NP, 1), jnp.float32)),
)

_node_call = pl.pallas_call(
    _node_body,
    out_shape=jax.ShapeDtypeStruct((NP, H), jnp.float32),
)

_final_call = pl.pallas_call(
    _final_body,
    out_shape=jax.ShapeDtypeStruct((1, 4), jnp.float32),
)


def kernel(x, edge_index, edge_attr, batch, params):
    p = params
    loop = jnp.arange(N, dtype=jnp.int32)
    npad = EP - (E + N)
    src = jnp.concatenate([
        edge_index[0].astype(jnp.int32), loop,
        jnp.full((npad,), DUMMY_SRC, jnp.int32)]).reshape(NW, NCH, CH)
    dst = jnp.concatenate([
        edge_index[1].astype(jnp.int32), loop,
        jnp.full((npad,), DUMMY_DST, jnp.int32)]).reshape(NW, NCH, CH)
    x_pad = jnp.pad(x, ((0, NP - N), (0, 0)))
    zeros2 = jnp.zeros((NP, H), jnp.float32)
    zeros1 = jnp.zeros((NP,), jnp.float32)
    ones_c = jnp.ones((CH,), jnp.float32)

    deg_kernel, round_kernel = _sc_kernels()
    deg2 = deg_kernel(dst, ones_c, zeros1)
    d0 = deg2[0].reshape(NP, 1)
    d1 = deg2[1].reshape(NP, 1)
    t, dinv = _pre_call(x_pad, p["Wi"], d0, d1)

    biases = [p["bi"], p["b0"], p["b1"], p["b2"], p["b3"]]
    weights = [p["W0"], p["W1"], p["W2"], p["W3"]]
    probs = None
    for i in range(5):
        a = round_kernel(t, src, dst, zeros2)
        a0, a1 = a[0], a[1]
        if i < 4:
            t = _node_call(a0, a1, dinv, biases[i].reshape(1, H), weights[i])
        else:
            probs = _final_call(
                a0, a1, dinv, biases[4].reshape(1, H),
                p["Wd1"], p["bd1"].reshape(1, H // 2),
                p["Wd2"], p["bd2"].reshape(1, 4))
    return (probs, edge_attr)


# speedup vs baseline: 2.5982x; 2.5982x over previous
import functools

import jax
import jax.numpy as jnp
from jax import lax
from jax.experimental import pallas as pl
from jax.experimental.pallas import tpu as pltpu
from jax.experimental.pallas import tpu_sc as plsc

N = 10000
E = 320000
D_IN = 128
H = 32

NP = 10240
DUMMY_DST = N
DUMMY_SRC = N + 1

NW = 32
CH = 128
NCH = 80
NBUF = 4
EP = NW * NCH * CH
ROWS_PER_TILE = NP // 16


@functools.cache
def _sc_kernels():
    mesh = plsc.VectorSubcoreMesh(core_axis_name="c", subcore_axis_name="s")
    params = pltpu.CompilerParams(use_tc_tiling_on_sc=False)

    @functools.partial(
        pl.kernel,
        out_type=jax.ShapeDtypeStruct((2, NP), jnp.float32),
        mesh=mesh,
        compiler_params=params,
        scratch_types=[
            pltpu.VMEM((NCH, CH), jnp.int32),
            pltpu.VMEM((CH,), jnp.float32),
            pltpu.VMEM_SHARED((NP,), jnp.float32),
            [pltpu.SemaphoreType.DMA] * NBUF,
        ],
    )
    def deg_kernel(didx_hbm, ones_hbm, zeros1_hbm, deg_out, didx_v, ones_v,
                   dacc, ssem):
        cid = lax.axis_index("c")
        sid = lax.axis_index("s")
        wid = sid * 2 + cid
        lo = sid * ROWS_PER_TILE
        pltpu.sync_copy(didx_hbm.at[wid], didx_v)
        pltpu.sync_copy(ones_hbm, ones_v)
        pltpu.sync_copy(zeros1_hbm.at[pl.ds(lo, ROWS_PER_TILE)],
                        dacc.at[pl.ds(lo, ROWS_PER_TILE)])
        plsc.subcore_barrier()

        for b in range(NBUF):
            pltpu.async_copy(ones_v, dacc.at[didx_v.at[b]], ssem[b], add=True)

        def body(it, carry):
            j = it * NBUF
            for b in range(NBUF):
                pltpu.make_async_copy(ones_v, dacc.at[didx_v.at[j + b]],
                                      ssem[b]).wait()
                pltpu.async_copy(ones_v, dacc.at[didx_v.at[j + NBUF + b]],
                                 ssem[b], add=True)
            return carry

        lax.fori_loop(0, NCH // NBUF - 1, body, 0)
        for b in range(NBUF):
            pltpu.make_async_copy(ones_v, dacc.at[didx_v.at[NCH - NBUF + b]],
                                  ssem[b]).wait()
        plsc.subcore_barrier()
        pltpu.sync_copy(dacc.at[pl.ds(lo, ROWS_PER_TILE)],
                        deg_out.at[cid, pl.ds(lo, ROWS_PER_TILE)])

    @functools.partial(
        pl.kernel,
        out_type=jax.ShapeDtypeStruct((2, NP, H), jnp.float32),
        mesh=mesh,
        compiler_params=params,
        scratch_types=[
            pltpu.VMEM((NCH, CH), jnp.int32),
            pltpu.VMEM((NCH, CH), jnp.int32),
            [pltpu.VMEM((CH, H), jnp.float32)] * NBUF,
            pltpu.VMEM_SHARED((NP, H), jnp.float32),
            pltpu.VMEM_SHARED((NP, H), jnp.float32),
            [pltpu.SemaphoreType.DMA] * NBUF,
            [pltpu.SemaphoreType.DMA] * NBUF,
        ],
    )
    def round_kernel(tab_hbm, sidx_hbm, didx_hbm, zeros2_hbm, out_hbm,
                     sidx_v, didx_v, bufs, tab_sh, acc, gsem, ssem):
        cid = lax.axis_index("c")
        sid = lax.axis_index("s")
        wid = sid * 2 + cid
        lo = sid * ROWS_PER_TILE
        pltpu.sync_copy(sidx_hbm.at[wid], sidx_v)
        pltpu.sync_copy(didx_hbm.at[wid], didx_v)
        pltpu.sync_copy(tab_hbm.at[pl.ds(lo, ROWS_PER_TILE)],
                        tab_sh.at[pl.ds(lo, ROWS_PER_TILE)])
        pltpu.sync_copy(zeros2_hbm.at[pl.ds(lo, ROWS_PER_TILE)],
                        acc.at[pl.ds(lo, ROWS_PER_TILE)])
        plsc.subcore_barrier()

        for b in range(NBUF):
            pltpu.async_copy(tab_sh.at[sidx_v.at[b]], bufs[b], gsem[b])

        def body(it, carry):
            j = it * NBUF
            for b in range(NBUF):
                pltpu.make_async_copy(tab_sh.at[sidx_v.at[j + b]],
                                      bufs[b], gsem[b]).wait()
                pltpu.async_copy(bufs[b], acc.at[didx_v.at[j + b]],
                                 ssem[b], add=True)
            for b in range(NBUF):
                pltpu.make_async_copy(bufs[b], acc.at[didx_v.at[j + b]],
                                      ssem[b]).wait()
                pltpu.async_copy(tab_sh.at[sidx_v.at[j + NBUF + b]],
                                 bufs[b], gsem[b])
            return carry

        lax.fori_loop(0, NCH // NBUF - 1, body, 0)
        j = NCH - NBUF
        for b in range(NBUF):
            pltpu.make_async_copy(tab_sh.at[sidx_v.at[j + b]],
                                  bufs[b], gsem[b]).wait()
            pltpu.async_copy(bufs[b], acc.at[didx_v.at[j + b]],
                             ssem[b], add=True)
        for b in range(NBUF):
            pltpu.make_async_copy(bufs[b], acc.at[didx_v.at[j + b]],
                                  ssem[b]).wait()
        plsc.subcore_barrier()
        pltpu.sync_copy(acc.at[pl.ds(lo, ROWS_PER_TILE)],
                        out_hbm.at[cid, pl.ds(lo, ROWS_PER_TILE)])

    return deg_kernel, round_kernel


def _pre_body(x_ref, wi_ref, deg2_ref, t_ref, dinv_ref):
    deg = deg2_ref[0] + deg2_ref[1] + 1.0
    dinv = lax.rsqrt(deg)
    m0 = jnp.dot(x_ref[...], wi_ref[...], preferred_element_type=jnp.float32)
    t_ref[...] = m0 * dinv
    dinv_ref[...] = dinv


def _node_body(a2_ref, tin_ref, dinv_ref, b_ref, w_ref, t_ref):
    dinv = dinv_ref[...]
    s = a2_ref[0] + a2_ref[1] + tin_ref[...]
    h = jnp.maximum(s * dinv + b_ref[...], 0.0)
    t_ref[...] = jnp.dot(h, w_ref[...], preferred_element_type=jnp.float32) * dinv


def _final_body(a2_ref, tin_ref, dinv_ref, b_ref, wd1_ref, bd1_ref,
                wd2_ref, bd2_ref, out_ref):
    s = a2_ref[0] + a2_ref[1] + tin_ref[...]
    h = jnp.maximum(s * dinv_ref[...] + b_ref[...], 0.0)
    rows = lax.broadcasted_iota(jnp.int32, (NP, H), 0)
    hm = jnp.where(rows < N, h, -jnp.inf)
    z = jnp.max(hm, axis=0, keepdims=True)
    z2 = jnp.maximum(
        jnp.dot(z, wd1_ref[...], preferred_element_type=jnp.float32) + bd1_ref[...],
        0.0)
    out_ref[...] = (jnp.dot(z2, wd2_ref[...], preferred_element_type=jnp.float32)
                    + bd2_ref[...])


_pre_call = pl.pallas_call(
    _pre_body,
    out_shape=(jax.ShapeDtypeStruct((NP, H), jnp.float32),
               jax.ShapeDtypeStruct((NP, 1), jnp.float32)),
)

_node_call = pl.pallas_call(
    _node_body,
    out_shape=jax.ShapeDtypeStruct((NP, H), jnp.float32),
)

_final_call = pl.pallas_call(
    _final_body,
    out_shape=jax.ShapeDtypeStruct((1, 4), jnp.float32),
)


def kernel(x, edge_index, edge_attr, batch, params):
    p = params
    npad = EP - E
    src = jnp.concatenate([
        edge_index[0].astype(jnp.int32),
        jnp.full((npad,), DUMMY_SRC, jnp.int32)]).reshape(NW, NCH, CH)
    dst = jnp.concatenate([
        edge_index[1].astype(jnp.int32),
        jnp.full((npad,), DUMMY_DST, jnp.int32)]).reshape(NW, NCH, CH)
    x_pad = jnp.pad(x, ((0, NP - N), (0, 0)))
    zeros1 = jnp.zeros((NP,), jnp.float32)
    zeros2 = jnp.zeros((NP, H), jnp.float32)
    ones_c = jnp.ones((CH,), jnp.float32)

    deg_kernel, round_kernel = _sc_kernels()
    deg2 = deg_kernel(dst, ones_c, zeros1)
    t, dinv = _pre_call(x_pad, p["Wi"], deg2.reshape(2, NP, 1))

    biases = [p["bi"], p["b0"], p["b1"], p["b2"], p["b3"]]
    weights = [p["W0"], p["W1"], p["W2"], p["W3"]]
    probs = None
    for i in range(5):
        a = round_kernel(t, src, dst, zeros2)
        if i < 4:
            t = _node_call(a, t, dinv, biases[i].reshape(1, H), weights[i])
        else:
            probs = _final_call(
                a, t, dinv, biases[4].reshape(1, H),
                p["Wd1"], p["bd1"].reshape(1, H // 2),
                p["Wd2"], p["bd2"].reshape(1, 4))
    return (probs, edge_attr)


# speedup vs baseline: 3.6748x; 1.4144x over previous
import functools

import jax
import jax.numpy as jnp
from jax import lax
from jax.experimental import pallas as pl
from jax.experimental.pallas import tpu as pltpu
from jax.experimental.pallas import tpu_sc as plsc

N = 10000
E = 320000
D_IN = 128
H = 32

NP = 10240

NW = 32
CH = 125
NCH = 80
NBUF = 4
ROWS_PER_TILE = NP // 16


@functools.cache
def _sc_kernels():
    mesh = plsc.VectorSubcoreMesh(core_axis_name="c", subcore_axis_name="s")
    params = pltpu.CompilerParams(use_tc_tiling_on_sc=False)

    @functools.partial(
        pl.kernel,
        out_type=jax.ShapeDtypeStruct((2, NP), jnp.float32),
        mesh=mesh,
        compiler_params=params,
        scratch_types=[
            pltpu.VMEM((NCH, CH), jnp.int32),
            pltpu.VMEM((CH,), jnp.float32),
            pltpu.VMEM_SHARED((NP,), jnp.float32),
            [pltpu.SemaphoreType.DMA] * NBUF,
        ],
    )
    def deg_kernel(didx_hbm, ones_hbm, zeros1_hbm, deg_out, didx_v, ones_v,
                   dacc, ssem):
        cid = lax.axis_index("c")
        sid = lax.axis_index("s")
        wid = sid * 2 + cid
        lo = sid * ROWS_PER_TILE
        pltpu.sync_copy(didx_hbm.at[wid], didx_v)
        pltpu.sync_copy(ones_hbm, ones_v)
        pltpu.sync_copy(zeros1_hbm.at[pl.ds(lo, ROWS_PER_TILE)],
                        dacc.at[pl.ds(lo, ROWS_PER_TILE)])
        plsc.subcore_barrier()

        for b in range(NBUF):
            pltpu.async_copy(ones_v, dacc.at[didx_v.at[b]], ssem[b], add=True)

        def body(it, carry):
            j = it * NBUF
            for b in range(NBUF):
                pltpu.make_async_copy(ones_v, dacc.at[didx_v.at[j + b]],
                                      ssem[b]).wait()
                pltpu.async_copy(ones_v, dacc.at[didx_v.at[j + NBUF + b]],
                                 ssem[b], add=True)
            return carry

        lax.fori_loop(0, NCH // NBUF - 1, body, 0)
        for b in range(NBUF):
            pltpu.make_async_copy(ones_v, dacc.at[didx_v.at[NCH - NBUF + b]],
                                  ssem[b]).wait()
        plsc.subcore_barrier()
        pltpu.sync_copy(dacc.at[pl.ds(lo, ROWS_PER_TILE)],
                        deg_out.at[cid, pl.ds(lo, ROWS_PER_TILE)])

    @functools.partial(
        pl.kernel,
        out_type=jax.ShapeDtypeStruct((2, NP, H), jnp.float32),
        mesh=mesh,
        compiler_params=params,
        scratch_types=[
            pltpu.VMEM((NCH, CH), jnp.int32),
            pltpu.VMEM((NCH, CH), jnp.int32),
            [pltpu.VMEM((CH, H), jnp.float32)] * NBUF,
            pltpu.VMEM_SHARED((NP, H), jnp.float32),
            pltpu.VMEM_SHARED((NP, H), jnp.float32),
            [pltpu.SemaphoreType.DMA] * NBUF,
            [pltpu.SemaphoreType.DMA] * NBUF,
        ],
    )
    def round_kernel(tab_hbm, sidx_hbm, didx_hbm, zeros2_hbm, out_hbm,
                     sidx_v, didx_v, bufs, tab_sh, acc, gsem, ssem):
        cid = lax.axis_index("c")
        sid = lax.axis_index("s")
        wid = sid * 2 + cid
        lo = sid * ROWS_PER_TILE
        pltpu.sync_copy(sidx_hbm.at[wid], sidx_v)
        pltpu.sync_copy(didx_hbm.at[wid], didx_v)
        pltpu.sync_copy(tab_hbm.at[pl.ds(lo, ROWS_PER_TILE)],
                        tab_sh.at[pl.ds(lo, ROWS_PER_TILE)])
        pltpu.sync_copy(zeros2_hbm.at[pl.ds(lo, ROWS_PER_TILE)],
                        acc.at[pl.ds(lo, ROWS_PER_TILE)])
        plsc.subcore_barrier()

        for b in range(NBUF):
            pltpu.async_copy(tab_sh.at[sidx_v.at[b]], bufs[b], gsem[b])

        def body(it, carry):
            j = it * NBUF
            for b in range(NBUF):
                pltpu.make_async_copy(tab_sh.at[sidx_v.at[j + b]],
                                      bufs[b], gsem[b]).wait()
                pltpu.async_copy(bufs[b], acc.at[didx_v.at[j + b]],
                                 ssem[b], add=True)
            for b in range(NBUF):
                pltpu.make_async_copy(bufs[b], acc.at[didx_v.at[j + b]],
                                      ssem[b]).wait()
                pltpu.async_copy(tab_sh.at[sidx_v.at[j + NBUF + b]],
                                 bufs[b], gsem[b])
            return carry

        lax.fori_loop(0, NCH // NBUF - 1, body, 0)
        j = NCH - NBUF
        for b in range(NBUF):
            pltpu.make_async_copy(tab_sh.at[sidx_v.at[j + b]],
                                  bufs[b], gsem[b]).wait()
            pltpu.async_copy(bufs[b], acc.at[didx_v.at[j + b]],
                             ssem[b], add=True)
        for b in range(NBUF):
            pltpu.make_async_copy(bufs[b], acc.at[didx_v.at[j + b]],
                                  ssem[b]).wait()
        plsc.subcore_barrier()
        pltpu.sync_copy(acc.at[pl.ds(lo, ROWS_PER_TILE)],
                        out_hbm.at[cid, pl.ds(lo, ROWS_PER_TILE)])

    return deg_kernel, round_kernel


NP4 = NP // 4
NROW4 = N // 4


def _pre_body(x4_ref, wibd_ref, deg4_ref, sel_ref, t_ref, dinv_ref):
    deg = deg4_ref[0] + deg4_ref[1] + 1.0
    dinv4 = lax.rsqrt(deg)
    dinv = jnp.dot(dinv4, sel_ref[...], preferred_element_type=jnp.float32)
    m0 = jnp.dot(x4_ref[...], wibd_ref[...], preferred_element_type=jnp.float32)
    t_ref[...] = m0 * dinv
    dinv_ref[...] = dinv


def _node_body(a2_ref, tin_ref, dinv_ref, b_ref, wbd_ref, t_ref):
    dinv = dinv_ref[...]
    s = a2_ref[0] + a2_ref[1] + tin_ref[...]
    h = jnp.maximum(s * dinv + b_ref[...], 0.0)
    t_ref[...] = jnp.dot(h, wbd_ref[...], preferred_element_type=jnp.float32) * dinv


def _final_body(a2_ref, tin_ref, dinv_ref, b_ref, wd1_ref, bd1_ref,
                wd2_ref, bd2_ref, out_ref):
    s = a2_ref[0] + a2_ref[1] + tin_ref[...]
    h = jnp.maximum(s * dinv_ref[...] + b_ref[...], 0.0)
    rows = lax.broadcasted_iota(jnp.int32, (NP4, 128), 0)
    hm = jnp.where(rows < NROW4, h, -jnp.inf)
    z4 = jnp.max(hm, axis=0, keepdims=True)
    z = jnp.maximum(jnp.maximum(z4[:, 0:H], z4[:, H:2 * H]),
                    jnp.maximum(z4[:, 2 * H:3 * H], z4[:, 3 * H:4 * H]))
    z2 = jnp.maximum(
        jnp.dot(z, wd1_ref[...], preferred_element_type=jnp.float32) + bd1_ref[...],
        0.0)
    out_ref[...] = (jnp.dot(z2, wd2_ref[...], preferred_element_type=jnp.float32)
                    + bd2_ref[...])


_pre_call = pl.pallas_call(
    _pre_body,
    out_shape=(jax.ShapeDtypeStruct((NP4, 128), jnp.float32),
               jax.ShapeDtypeStruct((NP4, 128), jnp.float32)),
)

_node_call = pl.pallas_call(
    _node_body,
    out_shape=jax.ShapeDtypeStruct((NP4, 128), jnp.float32),
)

_final_call = pl.pallas_call(
    _final_body,
    out_shape=jax.ShapeDtypeStruct((1, 4), jnp.float32),
)


def _blockdiag4(w):
    k, h = w.shape
    out = jnp.zeros((4 * k, 4 * h), w.dtype)
    for q in range(4):
        out = lax.dynamic_update_slice(out, w, (q * k, q * h))
    return out


def kernel(x, edge_index, edge_attr, batch, params):
    p = params
    src = edge_index[0].astype(jnp.int32).reshape(NW, NCH, CH)
    dst = edge_index[1].astype(jnp.int32).reshape(NW, NCH, CH)
    x_pad = jnp.pad(x, ((0, NP - N), (0, 0)))
    zeros1 = jnp.zeros((NP,), jnp.float32)
    zeros2 = jnp.zeros((NP, H), jnp.float32)
    ones_c = jnp.ones((CH,), jnp.float32)

    sel = jnp.kron(jnp.eye(4, dtype=jnp.float32), jnp.ones((1, H), jnp.float32))

    deg_kernel, round_kernel = _sc_kernels()
    deg2 = deg_kernel(dst, ones_c, zeros1)
    t, dinv = _pre_call(x_pad.reshape(NP4, 4 * D_IN), _blockdiag4(p["Wi"]),
                        deg2.reshape(2, NP4, 4), sel)

    biases = [p["bi"], p["b0"], p["b1"], p["b2"], p["b3"]]
    weights = [p["W0"], p["W1"], p["W2"], p["W3"]]
    probs = None
    for i in range(5):
        a = round_kernel(t.reshape(NP, H), src, dst, zeros2)
        a4 = a.reshape(2, NP4, 128)
        if i < 4:
            t = _node_call(a4, t, dinv, jnp.tile(biases[i], 4).reshape(1, 128),
                           _blockdiag4(weights[i]))
        else:
            probs = _final_call(
                a4, t, dinv, jnp.tile(biases[4], 4).reshape(1, 128),
                p["Wd1"], p["bd1"].reshape(1, H // 2),
                p["Wd2"], p["bd2"].reshape(1, 4))
    return (probs, edge_attr)
